# baseline (speedup 1.0000x reference)
import jax
import jax.numpy as jnp
from jax.experimental import pallas as pl

_BI = 16


def _pos_kernel(col_ref, row_ref, out_ref):
    c2, bi, w = out_ref.shape
    F = c2 // 2
    colT = col_ref[:].T
    rowT = row_ref[:].T
    out_ref[0:F] = jnp.broadcast_to(colT[:, None, :], (F, bi, w))
    out_ref[F:c2] = jnp.broadcast_to(rowT[:, :, None], (F, bi, w))


def kernel(image_tensor, row_embed, col_embed):
    h, w = image_tensor.shape[-2], image_tensor.shape[-1]
    F = row_embed.shape[1]
    out = pl.pallas_call(
        _pos_kernel,
        grid=(h // _BI,),
        in_specs=[
            pl.BlockSpec((w, F), lambda b: (0, 0)),
            pl.BlockSpec((_BI, F), lambda b: (b, 0)),
        ],
        out_specs=pl.BlockSpec((2 * F, _BI, w), lambda b: (0, b, 0)),
        out_shape=jax.ShapeDtypeStruct((2 * F, h, w), jnp.float32),
    )(col_embed[:w], row_embed[:h])
    return out[None]


# speedup vs baseline: 1.0023x; 1.0023x over previous
import jax
import jax.numpy as jnp
from jax.experimental import pallas as pl

_BC = 32


def _pos_kernel(col_ref, row_ref, out_ref):
    bc, h, w = out_ref.shape
    nb_half = pl.num_programs(0) // 2
    b = pl.program_id(0)
    for k in range(2 * nb_half):
        @pl.when(b == k)
        def _(k=k):
            if k < nb_half:
                slab = col_ref[:].T[k * bc:(k + 1) * bc, :]
                out_ref[...] = jnp.broadcast_to(slab[:, None, :], (bc, h, w))
            else:
                kk = k - nb_half
                slab = row_ref[:].T[kk * bc:(kk + 1) * bc, :]
                out_ref[...] = jnp.broadcast_to(slab[:, :, None], (bc, h, w))


def kernel(image_tensor, row_embed, col_embed):
    h, w = image_tensor.shape[-2], image_tensor.shape[-1]
    F = row_embed.shape[1]
    out = pl.pallas_call(
        _pos_kernel,
        grid=(2 * F // _BC,),
        in_specs=[
            pl.BlockSpec((w, F), lambda b: (0, 0)),
            pl.BlockSpec((h, F), lambda b: (0, 0)),
        ],
        out_specs=pl.BlockSpec((_BC, h, w), lambda b: (b, 0, 0)),
        out_shape=jax.ShapeDtypeStruct((2 * F, h, w), jnp.float32),
    )(col_embed[:w], row_embed[:h])
    return out[None]


# speedup vs baseline: 1.1218x; 1.1192x over previous
import jax
import jax.numpy as jnp
from jax.experimental import pallas as pl

_BC = 64


def _pos_kernel(col_ref, row_ref, out_ref):
    bc, h, w = out_ref.shape
    nb_half = pl.num_programs(0) // 2
    b = pl.program_id(0)
    for k in range(2 * nb_half):
        @pl.when(b == k)
        def _(k=k):
            if k < nb_half:
                slab = col_ref[:].T[k * bc:(k + 1) * bc, :]
                out_ref[...] = jnp.broadcast_to(slab[:, None, :], (bc, h, w))
            else:
                kk = k - nb_half
                slab = row_ref[:].T[kk * bc:(kk + 1) * bc, :]
                out_ref[...] = jnp.broadcast_to(slab[:, :, None], (bc, h, w))


def kernel(image_tensor, row_embed, col_embed):
    h, w = image_tensor.shape[-2], image_tensor.shape[-1]
    F = row_embed.shape[1]
    out = pl.pallas_call(
        _pos_kernel,
        grid=(2 * F // _BC,),
        in_specs=[
            pl.BlockSpec((w, F), lambda b: (0, 0)),
            pl.BlockSpec((h, F), lambda b: (0, 0)),
        ],
        out_specs=pl.BlockSpec((_BC, h, w), lambda b: (b, 0, 0)),
        out_shape=jax.ShapeDtypeStruct((2 * F, h, w), jnp.float32),
    )(col_embed[:w], row_embed[:h])
    return out[None]


# speedup vs baseline: 1.1944x; 1.0648x over previous
import jax
import jax.numpy as jnp
from jax.experimental import pallas as pl

_BC = 128


def _pos_kernel(col_ref, row_ref, out_ref):
    bc, h, w = out_ref.shape
    nb_half = pl.num_programs(0) // 2
    b = pl.program_id(0)
    for k in range(2 * nb_half):
        @pl.when(b == k)
        def _(k=k):
            if k < nb_half:
                slab = col_ref[:].T[k * bc:(k + 1) * bc, :]
                out_ref[...] = jnp.broadcast_to(slab[:, None, :], (bc, h, w))
            else:
                kk = k - nb_half
                slab = row_ref[:].T[kk * bc:(kk + 1) * bc, :]
                out_ref[...] = jnp.broadcast_to(slab[:, :, None], (bc, h, w))


def kernel(image_tensor, row_embed, col_embed):
    h, w = image_tensor.shape[-2], image_tensor.shape[-1]
    F = row_embed.shape[1]
    out = pl.pallas_call(
        _pos_kernel,
        grid=(2 * F // _BC,),
        in_specs=[
            pl.BlockSpec((w, F), lambda b: (0, 0)),
            pl.BlockSpec((h, F), lambda b: (0, 0)),
        ],
        out_specs=pl.BlockSpec((_BC, h, w), lambda b: (b, 0, 0)),
        out_shape=jax.ShapeDtypeStruct((2 * F, h, w), jnp.float32),
    )(col_embed[:w], row_embed[:h])
    return out[None]


# speedup vs baseline: 1.1998x; 1.0045x over previous
import jax
import jax.numpy as jnp
from jax.experimental import pallas as pl
from jax.experimental.pallas import tpu as pltpu

_BC = 128


def _pos_kernel(col_ref, row_ref, out_ref):
    bc, h, w = out_ref.shape
    nb_half = pl.num_programs(0) // 2
    b = pl.program_id(0)
    for k in range(2 * nb_half):
        @pl.when(b == k)
        def _(k=k):
            if k < nb_half:
                slab = col_ref[:].T[k * bc:(k + 1) * bc, :]
                out_ref[...] = jnp.broadcast_to(slab[:, None, :], (bc, h, w))
            else:
                kk = k - nb_half
                slab = row_ref[:].T[kk * bc:(kk + 1) * bc, :]
                out_ref[...] = jnp.broadcast_to(slab[:, :, None], (bc, h, w))


def kernel(image_tensor, row_embed, col_embed):
    h, w = image_tensor.shape[-2], image_tensor.shape[-1]
    F = row_embed.shape[1]
    out = pl.pallas_call(
        _pos_kernel,
        grid=(2 * F // _BC,),
        in_specs=[
            pl.BlockSpec((w, F), lambda b: (0, 0)),
            pl.BlockSpec((h, F), lambda b: (0, 0)),
        ],
        out_specs=pl.BlockSpec((_BC, h, w), lambda b: (b, 0, 0)),
        out_shape=jax.ShapeDtypeStruct((2 * F, h, w), jnp.float32),
        compiler_params=pltpu.CompilerParams(dimension_semantics=("parallel",)),
    )(col_embed[:w], row_embed[:h])
    return out[None]
